# baseline (speedup 1.0000x reference)
import functools

import numpy as np
import jax
import jax.numpy as jnp
from jax.experimental import pallas as pl
from jax.experimental.pallas import tpu as pltpu

_B = 32
_V = 100000
_K = 4
_VB = 2048
_NV = (_V + _VB - 1) // _VB
_NS = _VB // 128

_MAGIC = np.uint32(0x1BD11BDA)
_ROTS = ((13, 15, 26, 6), (17, 29, 16, 24))


@np.errstate(over="ignore")
def _np_threefry2x32(k1, k2, c0, c1):
    ks = [np.uint32(k1), np.uint32(k2)]
    ks.append(np.uint32(ks[0] ^ ks[1] ^ _MAGIC))
    x0 = np.uint32(np.uint32(c0) + ks[0])
    x1 = np.uint32(np.uint32(c1) + ks[1])
    for i in range(5):
        for r in _ROTS[i % 2]:
            x0 = np.uint32(x0 + x1)
            x1 = np.uint32((np.uint32(x1 << np.uint32(r))) |
                           (np.uint32(x1 >> np.uint32(32 - r))))
            x1 = np.uint32(x0 ^ x1)
        x0 = np.uint32(x0 + ks[(i + 1) % 3])
        x1 = np.uint32(x1 + ks[(i + 2) % 3] + np.uint32(i + 1))
    return x0, x1


_KEYS = [_np_threefry2x32(0, 42, 0, i) for i in (0, 1)]


def _rotl(x, r):
    return jax.lax.shift_left(x, r) | jax.lax.shift_right_logical(x, 32 - r)


def _threefry_bits(ks, x1):
    x0 = ks[0]
    for i in range(5):
        for r in _ROTS[i % 2]:
            x0 = x0 + x1
            x1 = _rotl(x1, r)
            x1 = x0 ^ x1
        x0 = x0 + ks[(i + 1) % 3]
        x1 = x1 + (ks[(i + 2) % 3] + jnp.int32(i + 1))
    return x0 ^ x1


def _sample_kernel(x0_ref, x1_ref, idxsum_ref, lpsum_ref,
                   base0, col0, zmax, colb, lxb, ssum):
    j = pl.program_id(0)

    @pl.when(j == 0)
    def _init():
        row = jax.lax.broadcasted_iota(jnp.int32, (_B, _VB), 0)
        col = jax.lax.broadcasted_iota(jnp.int32, (_B, _VB), 1)
        base0[...] = row * _V + col
        col0[...] = col
        zmax[...] = jnp.full((2 * _K, _B, 128), -jnp.inf, jnp.float32)
        colb[...] = jnp.zeros((2 * _K, _B, 128), jnp.float32)
        lxb[...] = jnp.zeros((2 * _K, _B, 128), jnp.float32)
        ssum[...] = jnp.zeros((2, _B, 128), jnp.float32)

    base = base0[...]
    valid = col0[...] < _V - j * _VB

    for d, x_ref in ((0, x0_ref), (1, x1_ref)):
        key = _KEYS[d]
        x = jnp.where(valid, x_ref[...], 0.0)
        lx = jnp.log(x)

        ss = ssum[d]
        for s in range(_NS):
            ss = ss + x[:, s * 128:(s + 1) * 128]
        ssum[d] = ss

        k1i = int(np.uint32(key[0]).astype(np.int32))
        k2i = int(np.uint32(key[1]).astype(np.int32))
        k3i = int((np.uint32(key[0]) ^ np.uint32(key[1]) ^ _MAGIC)
                  .astype(np.int32))
        ks = (jnp.int32(k1i), jnp.int32(k2i), jnp.int32(k3i))

        for kk in range(_K):
            off = int((np.uint32(kk * _B * _V) + np.uint32(key[1]))
                      .astype(np.int32))
            bits = _threefry_bits(ks, base + (j * _VB + off))
            fb = jax.lax.shift_right_logical(bits, 9) | jnp.int32(0x3F800000)
            u = jax.lax.bitcast_convert_type(fb, jnp.float32) - 1.0
            nl2 = jnp.log(-jnp.log(u))
            z = lx - nl2
            st = d * _K + kk
            zm, cb, lb = zmax[st], colb[st], lxb[st]
            for s in range(_NS):
                sl = slice(s * 128, (s + 1) * 128)
                zs = z[:, sl]
                upd = zs > zm
                tag = (j * _NS + s).astype(jnp.float32)
                zm = jnp.where(upd, zs, zm)
                cb = jnp.where(upd, tag, cb)
                lb = jnp.where(upd, lx[:, sl], lb)
            zmax[st], colb[st], lxb[st] = zm, cb, lb

    @pl.when(j == _NV - 1)
    def _finalize():
        lane = jax.lax.broadcasted_iota(
            jnp.int32, (_B, 128), 1).astype(jnp.float32)
        out_i = []
        out_l = []
        for d in range(2):
            srow = jnp.sum(ssum[d], axis=1)
            slog = jnp.sum(jnp.log(srow))
            for kk in range(_K):
                st = d * _K + kk
                zm, lb = zmax[st], lxb[st]
                colv = colb[st] * 128.0 + lane
                m = jnp.max(zm, axis=1, keepdims=True)
                csel = jnp.min(jnp.where(zm == m, colv, jnp.float32(2.0**31)),
                               axis=1, keepdims=True)
                lsel = jnp.max(jnp.where(colv == csel, lb, -jnp.inf), axis=1)
                out_i.append(jnp.sum(csel))
                out_l.append(jnp.sum(lsel) - slog)
        idxsum_ref[0] = jnp.stack(out_i[:_K]).reshape(1, _K)
        idxsum_ref[1] = jnp.stack(out_i[_K:]).reshape(1, _K)
        lpsum_ref[0] = jnp.stack(out_l[:_K]).reshape(1, _K)
        lpsum_ref[1] = jnp.stack(out_l[_K:]).reshape(1, _K)


def kernel(gt, x0, x1):
    del gt
    idxsum, lpsum = pl.pallas_call(
        _sample_kernel,
        grid=(_NV,),
        in_specs=[
            pl.BlockSpec((_B, _VB), lambda j: (0, j)),
            pl.BlockSpec((_B, _VB), lambda j: (0, j)),
        ],
        out_specs=[
            pl.BlockSpec((2, 1, _K), lambda j: (0, 0, 0)),
            pl.BlockSpec((2, 1, _K), lambda j: (0, 0, 0)),
        ],
        out_shape=[
            jax.ShapeDtypeStruct((2, 1, _K), jnp.float32),
            jax.ShapeDtypeStruct((2, 1, _K), jnp.float32),
        ],
        scratch_shapes=[
            pltpu.VMEM((_B, _VB), jnp.int32),
            pltpu.VMEM((_B, _VB), jnp.int32),
            pltpu.VMEM((2 * _K, _B, 128), jnp.float32),
            pltpu.VMEM((2 * _K, _B, 128), jnp.float32),
            pltpu.VMEM((2 * _K, _B, 128), jnp.float32),
            pltpu.VMEM((2, _B, 128), jnp.float32),
        ],
    )(x0, x1)
    results = idxsum[0, 0] / _B + idxsum[1, 0] / _B
    log_p = lpsum[:, 0]
    return results, log_p


# speedup vs baseline: 1.0375x; 1.0375x over previous
import functools

import numpy as np
import jax
import jax.numpy as jnp
from jax.experimental import pallas as pl
from jax.experimental.pallas import tpu as pltpu

_B = 32
_V = 100000
_K = 4
_VB = 1792
_NH = 2
_NV = (_V + _VB * _NH - 1) // (_VB * _NH)
_NS = _VB // 128

_MAGIC = np.uint32(0x1BD11BDA)
_ROTS = ((13, 15, 26, 6), (17, 29, 16, 24))


@np.errstate(over="ignore")
def _np_threefry2x32(k1, k2, c0, c1):
    ks = [np.uint32(k1), np.uint32(k2)]
    ks.append(np.uint32(ks[0] ^ ks[1] ^ _MAGIC))
    x0 = np.uint32(np.uint32(c0) + ks[0])
    x1 = np.uint32(np.uint32(c1) + ks[1])
    for i in range(5):
        for r in _ROTS[i % 2]:
            x0 = np.uint32(x0 + x1)
            x1 = np.uint32((np.uint32(x1 << np.uint32(r))) |
                           (np.uint32(x1 >> np.uint32(32 - r))))
            x1 = np.uint32(x0 ^ x1)
        x0 = np.uint32(x0 + ks[(i + 1) % 3])
        x1 = np.uint32(x1 + ks[(i + 2) % 3] + np.uint32(i + 1))
    return x0, x1


_KEYS = [_np_threefry2x32(0, 42, 0, i) for i in (0, 1)]


def _rotl(x, r):
    return jax.lax.shift_left(x, r) | jax.lax.shift_right_logical(x, 32 - r)


def _threefry_bits(ks, x1):
    x0 = ks[0]
    for i in range(5):
        for r in _ROTS[i % 2]:
            x0 = x0 + x1
            x1 = _rotl(x1, r)
            x1 = x0 ^ x1
        x0 = x0 + ks[(i + 1) % 3]
        x1 = x1 + (ks[(i + 2) % 3] + jnp.int32(i + 1))
    return x0 ^ x1


def _sample_kernel(x0_ref, x1_ref, idxsum_ref, lpsum_ref,
                   base0, col0, zmax, colb, lxb, ssum):
    j = pl.program_id(0)

    @pl.when(j == 0)
    def _init():
        row = jax.lax.broadcasted_iota(jnp.int32, (_B, _VB), 0)
        col = jax.lax.broadcasted_iota(jnp.int32, (_B, _VB), 1)
        base0[...] = row * _V + col
        col0[...] = col
        zmax[...] = jnp.full((2 * _K, _B, 128), -jnp.inf, jnp.float32)
        colb[...] = jnp.zeros((2 * _K, _B, 128), jnp.float32)
        lxb[...] = jnp.zeros((2 * _K, _B, 128), jnp.float32)
        ssum[...] = jnp.zeros((2, _B, 128), jnp.float32)

    base = base0[...]

    for h in range(_NH):
      hsl = slice(h * _VB, (h + 1) * _VB)
      jcol = (j * _NH + h) * _VB
      valid = col0[...] < _V - jcol
      for d, x_ref in ((0, x0_ref), (1, x1_ref)):
        key = _KEYS[d]
        x = jnp.where(valid, x_ref[:, hsl], 0.0)
        lx = jnp.log(x)

        ss = ssum[d]
        for s in range(_NS):
            ss = ss + x[:, s * 128:(s + 1) * 128]
        ssum[d] = ss

        k1i = int(np.uint32(key[0]).astype(np.int32))
        k2i = int(np.uint32(key[1]).astype(np.int32))
        k3i = int((np.uint32(key[0]) ^ np.uint32(key[1]) ^ _MAGIC)
                  .astype(np.int32))
        ks = (jnp.int32(k1i), jnp.int32(k2i), jnp.int32(k3i))

        for kk in range(_K):
            off = int((np.uint32(kk * _B * _V) + np.uint32(key[1]))
                      .astype(np.int32))
            bits = _threefry_bits(ks, base + (jcol + off))
            fb = jax.lax.shift_right_logical(bits, 9) | jnp.int32(0x3F800000)
            u = jax.lax.bitcast_convert_type(fb, jnp.float32) - 1.0
            nl2 = jnp.log(-jnp.log(u))
            z = lx - nl2
            st = d * _K + kk
            zm, cb, lb = zmax[st], colb[st], lxb[st]
            for s in range(_NS):
                sl = slice(s * 128, (s + 1) * 128)
                zs = z[:, sl]
                upd = zs > zm
                tag = ((j * _NH + h) * _NS + s).astype(jnp.float32)
                zm = jnp.where(upd, zs, zm)
                cb = jnp.where(upd, tag, cb)
                lb = jnp.where(upd, lx[:, sl], lb)
            zmax[st], colb[st], lxb[st] = zm, cb, lb

    @pl.when(j == _NV - 1)
    def _finalize():
        lane = jax.lax.broadcasted_iota(
            jnp.int32, (_B, 128), 1).astype(jnp.float32)
        out_i = []
        out_l = []
        for d in range(2):
            srow = jnp.sum(ssum[d], axis=1)
            slog = jnp.sum(jnp.log(srow))
            for kk in range(_K):
                st = d * _K + kk
                zm, lb = zmax[st], lxb[st]
                colv = colb[st] * 128.0 + lane
                m = jnp.max(zm, axis=1, keepdims=True)
                csel = jnp.min(jnp.where(zm == m, colv, jnp.float32(2.0**31)),
                               axis=1, keepdims=True)
                lsel = jnp.max(jnp.where(colv == csel, lb, -jnp.inf), axis=1)
                out_i.append(jnp.sum(csel))
                out_l.append(jnp.sum(lsel) - slog)
        idxsum_ref[0] = jnp.stack(out_i[:_K]).reshape(1, _K)
        idxsum_ref[1] = jnp.stack(out_i[_K:]).reshape(1, _K)
        lpsum_ref[0] = jnp.stack(out_l[:_K]).reshape(1, _K)
        lpsum_ref[1] = jnp.stack(out_l[_K:]).reshape(1, _K)


def kernel(gt, x0, x1):
    del gt
    idxsum, lpsum = pl.pallas_call(
        _sample_kernel,
        grid=(_NV,),
        in_specs=[
            pl.BlockSpec((_B, _VB * _NH), lambda j: (0, j)),
            pl.BlockSpec((_B, _VB * _NH), lambda j: (0, j)),
        ],
        out_specs=[
            pl.BlockSpec((2, 1, _K), lambda j: (0, 0, 0)),
            pl.BlockSpec((2, 1, _K), lambda j: (0, 0, 0)),
        ],
        out_shape=[
            jax.ShapeDtypeStruct((2, 1, _K), jnp.float32),
            jax.ShapeDtypeStruct((2, 1, _K), jnp.float32),
        ],
        scratch_shapes=[
            pltpu.VMEM((_B, _VB), jnp.int32),
            pltpu.VMEM((_B, _VB), jnp.int32),
            pltpu.VMEM((2 * _K, _B, 128), jnp.float32),
            pltpu.VMEM((2 * _K, _B, 128), jnp.float32),
            pltpu.VMEM((2 * _K, _B, 128), jnp.float32),
            pltpu.VMEM((2, _B, 128), jnp.float32),
        ],
    )(x0, x1)
    results = idxsum[0, 0] / _B + idxsum[1, 0] / _B
    log_p = lpsum[:, 0]
    return results, log_p


# speedup vs baseline: 1.0447x; 1.0069x over previous
import functools

import numpy as np
import jax
import jax.numpy as jnp
from jax.experimental import pallas as pl
from jax.experimental.pallas import tpu as pltpu

_B = 32
_V = 100000
_K = 4
_VB = 1792
_NV = (_V + _VB - 1) // _VB
_NS = _VB // 128

_MAGIC = np.uint32(0x1BD11BDA)
_ROTS = ((13, 15, 26, 6), (17, 29, 16, 24))


@np.errstate(over="ignore")
def _np_threefry2x32(k1, k2, c0, c1):
    ks = [np.uint32(k1), np.uint32(k2)]
    ks.append(np.uint32(ks[0] ^ ks[1] ^ _MAGIC))
    x0 = np.uint32(np.uint32(c0) + ks[0])
    x1 = np.uint32(np.uint32(c1) + ks[1])
    for i in range(5):
        for r in _ROTS[i % 2]:
            x0 = np.uint32(x0 + x1)
            x1 = np.uint32((np.uint32(x1 << np.uint32(r))) |
                           (np.uint32(x1 >> np.uint32(32 - r))))
            x1 = np.uint32(x0 ^ x1)
        x0 = np.uint32(x0 + ks[(i + 1) % 3])
        x1 = np.uint32(x1 + ks[(i + 2) % 3] + np.uint32(i + 1))
    return x0, x1


_KEYS = [_np_threefry2x32(0, 42, 0, i) for i in (0, 1)]


def _rotl(x, r):
    return jax.lax.shift_left(x, r) | jax.lax.shift_right_logical(x, 32 - r)


def _threefry_bits(ks, x1):
    x0 = ks[0]
    for i in range(5):
        for r in _ROTS[i % 2]:
            x0 = x0 + x1
            x1 = _rotl(x1, r)
            x1 = x0 ^ x1
        x0 = x0 + ks[(i + 1) % 3]
        x1 = x1 + (ks[(i + 2) % 3] + jnp.int32(i + 1))
    return x0 ^ x1


def _sample_kernel(x0_ref, x1_ref, idxsum_ref, lpsum_ref,
                   base0, col0, zmax, colb, lxb, ssum):
    j = pl.program_id(0)

    @pl.when(j == 0)
    def _init():
        row = jax.lax.broadcasted_iota(jnp.int32, (_B, _VB), 0)
        col = jax.lax.broadcasted_iota(jnp.int32, (_B, _VB), 1)
        base0[...] = row * _V + col
        col0[...] = col
        zmax[...] = jnp.full((2 * _K, _B, 128), -jnp.inf, jnp.float32)
        colb[...] = jnp.zeros((2 * _K, _B, 128), jnp.float32)
        lxb[...] = jnp.zeros((2 * _K, _B, 128), jnp.float32)
        ssum[...] = jnp.zeros((2, _B, 128), jnp.float32)

    base = base0[...]
    valid = col0[...] < _V - j * _VB

    for d, x_ref in ((0, x0_ref), (1, x1_ref)):
        key = _KEYS[d]
        x = jnp.where(valid, x_ref[...], 0.0)
        lx = jnp.log(x)

        ss = ssum[d]
        for s in range(_NS):
            ss = ss + x[:, s * 128:(s + 1) * 128]
        ssum[d] = ss

        k1i = int(np.uint32(key[0]).astype(np.int32))
        k2i = int(np.uint32(key[1]).astype(np.int32))
        k3i = int((np.uint32(key[0]) ^ np.uint32(key[1]) ^ _MAGIC)
                  .astype(np.int32))
        ks = (jnp.int32(k1i), jnp.int32(k2i), jnp.int32(k3i))

        for kk in range(_K):
            off = int((np.uint32(kk * _B * _V) + np.uint32(key[1]))
                      .astype(np.int32))
            bits = _threefry_bits(ks, base + (j * _VB + off))
            fb = jax.lax.shift_right_logical(bits, 9) | jnp.int32(0x3F800000)
            u = jax.lax.bitcast_convert_type(fb, jnp.float32) - 1.0
            nl2 = jnp.log(-jnp.log(u))
            z = lx - nl2
            st = d * _K + kk
            zm, cb, lb = zmax[st], colb[st], lxb[st]
            for s in range(_NS):
                sl = slice(s * 128, (s + 1) * 128)
                zs = z[:, sl]
                upd = zs > zm
                tag = (j * _NS + s).astype(jnp.float32)
                zm = jnp.where(upd, zs, zm)
                cb = jnp.where(upd, tag, cb)
                lb = jnp.where(upd, lx[:, sl], lb)
            zmax[st], colb[st], lxb[st] = zm, cb, lb

    @pl.when(j == _NV - 1)
    def _finalize():
        lane = jax.lax.broadcasted_iota(
            jnp.int32, (_B, 128), 1).astype(jnp.float32)
        out_i = []
        out_l = []
        for d in range(2):
            srow = jnp.sum(ssum[d], axis=1)
            slog = jnp.sum(jnp.log(srow))
            for kk in range(_K):
                st = d * _K + kk
                zm, lb = zmax[st], lxb[st]
                colv = colb[st] * 128.0 + lane
                m = jnp.max(zm, axis=1, keepdims=True)
                csel = jnp.min(jnp.where(zm == m, colv, jnp.float32(2.0**31)),
                               axis=1, keepdims=True)
                lsel = jnp.max(jnp.where(colv == csel, lb, -jnp.inf), axis=1)
                out_i.append(jnp.sum(csel))
                out_l.append(jnp.sum(lsel) - slog)
        idxsum_ref[0] = jnp.stack(out_i[:_K]).reshape(1, _K)
        idxsum_ref[1] = jnp.stack(out_i[_K:]).reshape(1, _K)
        lpsum_ref[0] = jnp.stack(out_l[:_K]).reshape(1, _K)
        lpsum_ref[1] = jnp.stack(out_l[_K:]).reshape(1, _K)


def kernel(gt, x0, x1):
    del gt
    idxsum, lpsum = pl.pallas_call(
        _sample_kernel,
        grid=(_NV,),
        in_specs=[
            pl.BlockSpec((_B, _VB), lambda j: (0, j)),
            pl.BlockSpec((_B, _VB), lambda j: (0, j)),
        ],
        out_specs=[
            pl.BlockSpec((2, 1, _K), lambda j: (0, 0, 0)),
            pl.BlockSpec((2, 1, _K), lambda j: (0, 0, 0)),
        ],
        out_shape=[
            jax.ShapeDtypeStruct((2, 1, _K), jnp.float32),
            jax.ShapeDtypeStruct((2, 1, _K), jnp.float32),
        ],
        scratch_shapes=[
            pltpu.VMEM((_B, _VB), jnp.int32),
            pltpu.VMEM((_B, _VB), jnp.int32),
            pltpu.VMEM((2 * _K, _B, 128), jnp.float32),
            pltpu.VMEM((2 * _K, _B, 128), jnp.float32),
            pltpu.VMEM((2 * _K, _B, 128), jnp.float32),
            pltpu.VMEM((2, _B, 128), jnp.float32),
        ],
    )(x0, x1)
    results = idxsum[0, 0] / _B + idxsum[1, 0] / _B
    log_p = lpsum[:, 0]
    return results, log_p
